# baseline (speedup 1.0000x reference)
import functools

import jax
import jax.numpy as jnp
from jax import lax
from jax.experimental import pallas as pl
from jax.experimental.pallas import tpu as pltpu
from jax.experimental.pallas import tpu_sc as plsc

_BATCH = 64
_TOKENS = 128
_DIM = 4096
_ROWS = _BATCH * _TOKENS

_NC = 2
_NS = 16
_NW = _NC * _NS
_B_PER_W = _ROWS // _NW
_CH = 8
_NCHUNK = _B_PER_W // _CH
_STAGE = _TOKENS // _NS


def _make_sc_lookup():
    mesh = plsc.VectorSubcoreMesh(core_axis_name="c", subcore_axis_name="s")

    @functools.partial(
        pl.kernel,
        mesh=mesh,
        out_type=jax.ShapeDtypeStruct((_ROWS, _DIM), jnp.float32),
        scratch_types=[
            pltpu.VMEM((_B_PER_W + 8,), jnp.int32),
            pltpu.VMEM((2, _CH, _DIM), jnp.float32),
            pltpu.VMEM_SHARED((_TOKENS, _DIM), jnp.float32),
            pltpu.SemaphoreType.DMA,
            pltpu.SemaphoreType.DMA,
            pltpu.SemaphoreType.DMA,
        ],
    )
    def sc_lookup(idx_hbm, table_hbm, out_hbm, idx_v, bufs, shared_tab,
                  csem, wsem0, wsem1):
        sid = lax.axis_index("s")
        wid = sid * _NC + lax.axis_index("c")
        base = wid * _B_PER_W
        pltpu.sync_copy(table_hbm.at[pl.ds(sid * _STAGE, _STAGE)],
                        shared_tab.at[pl.ds(sid * _STAGE, _STAGE)])
        pltpu.sync_copy(idx_hbm.at[pl.ds(base, _B_PER_W)],
                        idx_v.at[pl.ds(0, _B_PER_W)])
        plsc.subcore_barrier()

        wsems = (wsem0, wsem1)

        def fill(c, b):
            vec = idx_v[pl.ds(c * _CH, 16)]
            handles = []
            for k in range(_CH):
                handles.append(pltpu.async_copy(
                    shared_tab.at[vec[k]], bufs.at[b].at[k], csem))
            for h in handles:
                h.wait()

        def start_write(c, b):
            return pltpu.async_copy(
                bufs.at[b], out_hbm.at[pl.ds(base + c * _CH, _CH)], wsems[b])

        def wait_write(c, b):
            pltpu.make_async_copy(
                bufs.at[b], out_hbm.at[pl.ds(base + c * _CH, _CH)],
                wsems[b]).wait()

        fill(0, 0)
        start_write(0, 0)
        fill(1, 1)
        start_write(1, 1)

        def step(i, carry):
            for b in range(2):
                c = 2 + i * 2 + b
                wait_write(c - 2, b)
                fill(c, b)
                start_write(c, b)
            return carry

        lax.fori_loop(0, (_NCHUNK - 2) // 2, step, 0)
        wait_write(_NCHUNK - 2, 0)
        wait_write(_NCHUNK - 1, 1)

    return sc_lookup


_sc_lookup = _make_sc_lookup()


def kernel(indices, table):
    idx_flat = indices.reshape(_ROWS).astype(jnp.int32)
    out = _sc_lookup(idx_flat, table)
    return out.reshape(_BATCH, _TOKENS, _DIM)
